# baseline (speedup 1.0000x reference)
import functools

import jax
import jax.numpy as jnp
from jax import lax
from jax.experimental import pallas as pl
from jax.experimental.pallas import tpu as pltpu
from jax.experimental.pallas import tpu_sc as plsc

VOCAB = 1000000
EMBED = 64
HIDDEN = 64
B = 4096
T = 50
G3 = 3 * HIDDEN
B2 = B // 2
PW = 2 * EMBED
PG = 2 * G3

NC = 2
NS = 16
NW = NC * NS
CH2 = 400
KD = 16


def _gather_sc(emb, idx_even, idx_odd):
    TB2 = idx_even.shape[0]
    b_per_w = TB2 // NW
    n_ch = b_per_w // CH2
    mesh = plsc.VectorSubcoreMesh(core_axis_name="c", subcore_axis_name="s")

    @functools.partial(
        pl.kernel,
        out_type=(jax.ShapeDtypeStruct((TB2, EMBED), jnp.float32),
                  jax.ShapeDtypeStruct((TB2, EMBED), jnp.float32)),
        mesh=mesh,
        scratch_types=[
            pltpu.VMEM((CH2,), jnp.int32),
            pltpu.VMEM((CH2,), jnp.int32),
            pltpu.VMEM((CH2, EMBED), jnp.float32),
            pltpu.VMEM((CH2, EMBED), jnp.float32),
            pltpu.SemaphoreType.DMA,
        ],
    )
    def gather_kernel(table_hbm, ie_hbm, io_hbm, oe_hbm, oo_hbm,
                      ie_v, io_v, rows_e, rows_o, sem):
        wid = lax.axis_index("s") * NC + lax.axis_index("c")

        @pl.loop(0, n_ch)
        def _(c):
            base = wid * b_per_w + c * CH2
            pltpu.sync_copy(ie_hbm.at[pl.ds(base, CH2)], ie_v)
            pltpu.sync_copy(io_hbm.at[pl.ds(base, CH2)], io_v)

            @pl.loop(0, CH2, step=KD)
            def _(j0):
                ve = ie_v[pl.ds(j0, KD)]
                vo = io_v[pl.ds(j0, KD)]
                copies = []
                for j in range(KD):
                    copies.append(pltpu.async_copy(
                        table_hbm.at[pl.ds(ve[j], 1)],
                        rows_e.at[pl.ds(j0 + j, 1)], sem))
                    copies.append(pltpu.async_copy(
                        table_hbm.at[pl.ds(vo[j], 1)],
                        rows_o.at[pl.ds(j0 + j, 1)], sem))
                for cp in copies:
                    cp.wait()

            pltpu.sync_copy(rows_e, oe_hbm.at[pl.ds(base, CH2)])
            pltpu.sync_copy(rows_o, oo_hbm.at[pl.ds(base, CH2)])

    return gather_kernel(emb, idx_even, idx_odd)


def _gru_body(xe_hbm, xo_hbm, wih_ref, whh_ref, bih_ref, bhh_ref,
              fcw_ref, fcb_ref, out_ref,
              xe0, xo0, xe1, xo1, h_ref, sem0, sem1):
    pltpu.make_async_copy(xe_hbm.at[0], xe0, sem0).start()
    pltpu.make_async_copy(xo_hbm.at[0], xo0, sem0).start()
    pltpu.make_async_copy(xe_hbm.at[1], xe1, sem1).start()
    pltpu.make_async_copy(xo_hbm.at[1], xo1, sem1).start()
    h_ref[...] = jnp.zeros((B2, PW), jnp.float32)
    wih = wih_ref[...]
    whh = whh_ref[...]
    bih = bih_ref[...]
    bhh = bhh_ref[...]

    def gru_step(xt, h):
        gi = jnp.dot(xt, wih, preferred_element_type=jnp.float32) + bih
        gh = jnp.dot(h, whh, preferred_element_type=jnp.float32) + bhh
        r = jax.nn.sigmoid(gi[:, 0:128] + gh[:, 0:128])
        z = jax.nn.sigmoid(gi[:, 128:256] + gh[:, 128:256])
        n = jnp.tanh(gi[:, 256:384] + r * gh[:, 256:384])
        return (1.0 - z) * n + z * h

    def pair(i, carry):
        t0 = 2 * i
        pltpu.make_async_copy(xe_hbm.at[t0], xe0, sem0).wait()
        pltpu.make_async_copy(xo_hbm.at[t0], xo0, sem0).wait()
        xt = jnp.concatenate([xe0[...], xo0[...]], axis=1)
        h_ref[...] = gru_step(xt, h_ref[...])

        @pl.when(i < (T // 2) - 1)
        def _():
            pltpu.make_async_copy(xe_hbm.at[t0 + 2], xe0, sem0).start()
            pltpu.make_async_copy(xo_hbm.at[t0 + 2], xo0, sem0).start()

        pltpu.make_async_copy(xe_hbm.at[t0 + 1], xe1, sem1).wait()
        pltpu.make_async_copy(xo_hbm.at[t0 + 1], xo1, sem1).wait()
        xt = jnp.concatenate([xe1[...], xo1[...]], axis=1)
        h_ref[...] = gru_step(xt, h_ref[...])

        @pl.when(i < (T // 2) - 1)
        def _():
            pltpu.make_async_copy(xe_hbm.at[t0 + 3], xe1, sem1).start()
            pltpu.make_async_copy(xo_hbm.at[t0 + 3], xo1, sem1).start()

        return carry

    lax.fori_loop(0, T // 2, pair, 0)
    out_ref[...] = (
        jnp.dot(h_ref[...], fcw_ref[...], preferred_element_type=jnp.float32)
        + fcb_ref[...]
    )


def _gru_tc(xs_e, xs_o, wih2, whh2, bih2, bhh2, fcw2, fcb2):
    return pl.pallas_call(
        _gru_body,
        in_specs=[
            pl.BlockSpec(memory_space=pl.ANY),
            pl.BlockSpec(memory_space=pl.ANY),
            pl.BlockSpec(memory_space=pltpu.MemorySpace.VMEM),
            pl.BlockSpec(memory_space=pltpu.MemorySpace.VMEM),
            pl.BlockSpec(memory_space=pltpu.MemorySpace.VMEM),
            pl.BlockSpec(memory_space=pltpu.MemorySpace.VMEM),
            pl.BlockSpec(memory_space=pltpu.MemorySpace.VMEM),
            pl.BlockSpec(memory_space=pltpu.MemorySpace.VMEM),
        ],
        out_specs=pl.BlockSpec(memory_space=pltpu.MemorySpace.VMEM),
        out_shape=jax.ShapeDtypeStruct((B2, PW), jnp.float32),
        scratch_shapes=[
            pltpu.VMEM((B2, EMBED), jnp.float32),
            pltpu.VMEM((B2, EMBED), jnp.float32),
            pltpu.VMEM((B2, EMBED), jnp.float32),
            pltpu.VMEM((B2, EMBED), jnp.float32),
            pltpu.VMEM((B2, PW), jnp.float32),
            pltpu.SemaphoreType.DMA,
            pltpu.SemaphoreType.DMA,
        ],
    )(xs_e, xs_o, wih2, whh2, bih2, bhh2, fcw2, fcb2)


def _pair_weights(wT):
    z = jnp.zeros((EMBED, HIDDEN), wT.dtype)
    blocks = []
    for g in range(3):
        wg = wT[:, g * HIDDEN:(g + 1) * HIDDEN]
        top = jnp.concatenate([wg, z], axis=1)
        bot = jnp.concatenate([z, wg], axis=1)
        blocks.append(jnp.concatenate([top, bot], axis=0))
    return jnp.concatenate(blocks, axis=1)


def _pair_bias(b):
    return jnp.tile(b.reshape(3, 1, HIDDEN), (1, 2, 1)).reshape(1, PG)


def kernel(x, emb, w_ih, w_hh, b_ih, b_hh, fc_w, fc_b):
    idx = x.astype(jnp.int32).T.reshape(-1, 2)
    rows_e, rows_o = _gather_sc(emb, idx[:, 0], idx[:, 1])
    xs_e = rows_e.reshape(T, B2, EMBED)
    xs_o = rows_o.reshape(T, B2, EMBED)

    fcwT = fc_w.T
    z = jnp.zeros((HIDDEN, HIDDEN), fcwT.dtype)
    fcw2 = jnp.concatenate(
        [jnp.concatenate([fcwT, z], axis=1),
         jnp.concatenate([z, fcwT], axis=1)], axis=0)

    out2 = _gru_tc(
        xs_e, xs_o,
        _pair_weights(w_ih.T),
        _pair_weights(w_hh.T),
        _pair_bias(b_ih),
        _pair_bias(b_hh),
        fcw2,
        jnp.tile(fc_b.reshape(1, HIDDEN), (1, 2)),
    )
    return out2.reshape(B, HIDDEN)


# speedup vs baseline: 1.2086x; 1.2086x over previous
import functools

import jax
import jax.numpy as jnp
from jax import lax
from jax.experimental import pallas as pl
from jax.experimental.pallas import tpu as pltpu
from jax.experimental.pallas import tpu_sc as plsc

VOCAB = 1000000
EMBED = 64
HIDDEN = 64
B = 4096
T = 50
G3 = 3 * HIDDEN
B2 = B // 2
PW = 2 * EMBED
PG = 2 * G3

NC = 2
NS = 16
NW = NC * NS
CH2 = 400
KD = 16


def _gather_sc(emb, idx_even, idx_odd):
    TB2 = idx_even.shape[0]
    b_per_w = TB2 // NW
    n_ch = b_per_w // CH2
    mesh = plsc.VectorSubcoreMesh(core_axis_name="c", subcore_axis_name="s")

    @functools.partial(
        pl.kernel,
        out_type=(jax.ShapeDtypeStruct((TB2, EMBED), jnp.float32),
                  jax.ShapeDtypeStruct((TB2, EMBED), jnp.float32)),
        mesh=mesh,
        scratch_types=[
            pltpu.VMEM((CH2,), jnp.int32),
            pltpu.VMEM((CH2,), jnp.int32),
            pltpu.VMEM((CH2, EMBED), jnp.float32),
            pltpu.VMEM((CH2, EMBED), jnp.float32),
            pltpu.SemaphoreType.DMA,
        ],
    )
    def gather_kernel(table_hbm, ie_hbm, io_hbm, oe_hbm, oo_hbm,
                      ie_v, io_v, rows_e, rows_o, sem):
        wid = lax.axis_index("s") * NC + lax.axis_index("c")

        @pl.loop(0, n_ch)
        def _(c):
            base = wid * b_per_w + c * CH2
            pltpu.sync_copy(ie_hbm.at[pl.ds(base, CH2)], ie_v)
            pltpu.sync_copy(io_hbm.at[pl.ds(base, CH2)], io_v)

            @pl.loop(0, CH2, step=KD)
            def _(j0):
                ve = ie_v[pl.ds(j0, KD)]
                vo = io_v[pl.ds(j0, KD)]
                for j in range(KD):
                    pltpu.async_copy(
                        table_hbm.at[pl.ds(ve[j], 1)],
                        rows_e.at[pl.ds(j0 + j, 1)], sem)
                    pltpu.async_copy(
                        table_hbm.at[pl.ds(vo[j], 1)],
                        rows_o.at[pl.ds(j0 + j, 1)], sem)

            pltpu.make_async_copy(
                table_hbm.at[pl.ds(0, CH2)], rows_e, sem).wait()
            pltpu.make_async_copy(
                table_hbm.at[pl.ds(0, CH2)], rows_o, sem).wait()

            pltpu.sync_copy(rows_e, oe_hbm.at[pl.ds(base, CH2)])
            pltpu.sync_copy(rows_o, oo_hbm.at[pl.ds(base, CH2)])

    return gather_kernel(emb, idx_even, idx_odd)


def _gru_body(xe_hbm, xo_hbm, wih_ref, whh_ref, bih_ref, bhh_ref,
              fcw_ref, fcb_ref, out_ref,
              xe0, xo0, xe1, xo1, h_ref, sem0, sem1):
    pltpu.make_async_copy(xe_hbm.at[0], xe0, sem0).start()
    pltpu.make_async_copy(xo_hbm.at[0], xo0, sem0).start()
    pltpu.make_async_copy(xe_hbm.at[1], xe1, sem1).start()
    pltpu.make_async_copy(xo_hbm.at[1], xo1, sem1).start()
    h_ref[...] = jnp.zeros((B2, PW), jnp.float32)
    wih = wih_ref[...]
    whh = whh_ref[...]
    bih = bih_ref[...]
    bhh = bhh_ref[...]

    def gru_step(xt, h):
        gi = jnp.dot(xt, wih, preferred_element_type=jnp.float32) + bih
        gh = jnp.dot(h, whh, preferred_element_type=jnp.float32) + bhh
        r = jax.nn.sigmoid(gi[:, 0:128] + gh[:, 0:128])
        z = jax.nn.sigmoid(gi[:, 128:256] + gh[:, 128:256])
        n = jnp.tanh(gi[:, 256:384] + r * gh[:, 256:384])
        return (1.0 - z) * n + z * h

    def pair(i, carry):
        t0 = 2 * i
        pltpu.make_async_copy(xe_hbm.at[t0], xe0, sem0).wait()
        pltpu.make_async_copy(xo_hbm.at[t0], xo0, sem0).wait()
        xt = jnp.concatenate([xe0[...], xo0[...]], axis=1)
        h_ref[...] = gru_step(xt, h_ref[...])

        @pl.when(i < (T // 2) - 1)
        def _():
            pltpu.make_async_copy(xe_hbm.at[t0 + 2], xe0, sem0).start()
            pltpu.make_async_copy(xo_hbm.at[t0 + 2], xo0, sem0).start()

        pltpu.make_async_copy(xe_hbm.at[t0 + 1], xe1, sem1).wait()
        pltpu.make_async_copy(xo_hbm.at[t0 + 1], xo1, sem1).wait()
        xt = jnp.concatenate([xe1[...], xo1[...]], axis=1)
        h_ref[...] = gru_step(xt, h_ref[...])

        @pl.when(i < (T // 2) - 1)
        def _():
            pltpu.make_async_copy(xe_hbm.at[t0 + 3], xe1, sem1).start()
            pltpu.make_async_copy(xo_hbm.at[t0 + 3], xo1, sem1).start()

        return carry

    lax.fori_loop(0, T // 2, pair, 0)
    out_ref[...] = (
        jnp.dot(h_ref[...], fcw_ref[...], preferred_element_type=jnp.float32)
        + fcb_ref[...]
    )


def _gru_tc(xs_e, xs_o, wih2, whh2, bih2, bhh2, fcw2, fcb2):
    return pl.pallas_call(
        _gru_body,
        in_specs=[
            pl.BlockSpec(memory_space=pl.ANY),
            pl.BlockSpec(memory_space=pl.ANY),
            pl.BlockSpec(memory_space=pltpu.MemorySpace.VMEM),
            pl.BlockSpec(memory_space=pltpu.MemorySpace.VMEM),
            pl.BlockSpec(memory_space=pltpu.MemorySpace.VMEM),
            pl.BlockSpec(memory_space=pltpu.MemorySpace.VMEM),
            pl.BlockSpec(memory_space=pltpu.MemorySpace.VMEM),
            pl.BlockSpec(memory_space=pltpu.MemorySpace.VMEM),
        ],
        out_specs=pl.BlockSpec(memory_space=pltpu.MemorySpace.VMEM),
        out_shape=jax.ShapeDtypeStruct((B2, PW), jnp.float32),
        scratch_shapes=[
            pltpu.VMEM((B2, EMBED), jnp.float32),
            pltpu.VMEM((B2, EMBED), jnp.float32),
            pltpu.VMEM((B2, EMBED), jnp.float32),
            pltpu.VMEM((B2, EMBED), jnp.float32),
            pltpu.VMEM((B2, PW), jnp.float32),
            pltpu.SemaphoreType.DMA,
            pltpu.SemaphoreType.DMA,
        ],
    )(xs_e, xs_o, wih2, whh2, bih2, bhh2, fcw2, fcb2)


def _pair_weights(wT):
    z = jnp.zeros((EMBED, HIDDEN), wT.dtype)
    blocks = []
    for g in range(3):
        wg = wT[:, g * HIDDEN:(g + 1) * HIDDEN]
        top = jnp.concatenate([wg, z], axis=1)
        bot = jnp.concatenate([z, wg], axis=1)
        blocks.append(jnp.concatenate([top, bot], axis=0))
    return jnp.concatenate(blocks, axis=1)


def _pair_bias(b):
    return jnp.tile(b.reshape(3, 1, HIDDEN), (1, 2, 1)).reshape(1, PG)


def kernel(x, emb, w_ih, w_hh, b_ih, b_hh, fc_w, fc_b):
    idx = x.astype(jnp.int32).T.reshape(-1, 2)
    rows_e, rows_o = _gather_sc(emb, idx[:, 0], idx[:, 1])
    xs_e = rows_e.reshape(T, B2, EMBED)
    xs_o = rows_o.reshape(T, B2, EMBED)

    fcwT = fc_w.T
    z = jnp.zeros((HIDDEN, HIDDEN), fcwT.dtype)
    fcw2 = jnp.concatenate(
        [jnp.concatenate([fcwT, z], axis=1),
         jnp.concatenate([z, fcwT], axis=1)], axis=0)

    out2 = _gru_tc(
        xs_e, xs_o,
        _pair_weights(w_ih.T),
        _pair_weights(w_hh.T),
        _pair_bias(b_ih),
        _pair_bias(b_hh),
        fcw2,
        jnp.tile(fc_b.reshape(1, HIDDEN), (1, 2)),
    )
    return out2.reshape(B, HIDDEN)


# speedup vs baseline: 1.3803x; 1.1421x over previous
import functools

import jax
import jax.numpy as jnp
from jax import lax
from jax.experimental import pallas as pl
from jax.experimental.pallas import tpu as pltpu
from jax.experimental.pallas import tpu_sc as plsc

VOCAB = 1000000
EMBED = 64
HIDDEN = 64
B = 4096
T = 50
G3 = 3 * HIDDEN
B2 = B // 2
PW = 2 * EMBED
PG = 2 * G3

NC = 2
NS = 16
NW = NC * NS
CH2 = 400
KD = 16


def _gather_sc(emb, idx_even, idx_odd):
    TB2 = idx_even.shape[0]
    b_per_w = TB2 // NW
    n_ch = b_per_w // CH2
    mesh = plsc.VectorSubcoreMesh(core_axis_name="c", subcore_axis_name="s")

    @functools.partial(
        pl.kernel,
        out_type=(jax.ShapeDtypeStruct((TB2, EMBED), jnp.float32),
                  jax.ShapeDtypeStruct((TB2, EMBED), jnp.float32)),
        mesh=mesh,
        scratch_types=[
            pltpu.VMEM((CH2,), jnp.int32),
            pltpu.VMEM((CH2,), jnp.int32),
            pltpu.VMEM((CH2, EMBED), jnp.float32),
            pltpu.VMEM((CH2, EMBED), jnp.float32),
            pltpu.SemaphoreType.DMA,
        ],
    )
    def gather_kernel(table_hbm, ie_hbm, io_hbm, oe_hbm, oo_hbm,
                      ie_v, io_v, rows_e, rows_o, sem):
        wid = lax.axis_index("s") * NC + lax.axis_index("c")

        @pl.loop(0, n_ch)
        def _(c):
            base = wid * b_per_w + c * CH2
            pltpu.sync_copy(ie_hbm.at[pl.ds(base, CH2)], ie_v)
            pltpu.sync_copy(io_hbm.at[pl.ds(base, CH2)], io_v)

            @pl.loop(0, CH2, step=KD)
            def _(j0):
                ve = ie_v[pl.ds(j0, KD)]
                vo = io_v[pl.ds(j0, KD)]
                for j in range(KD):
                    pltpu.async_copy(
                        table_hbm.at[pl.ds(ve[j], 1)],
                        rows_e.at[pl.ds(j0 + j, 1)], sem)
                    pltpu.async_copy(
                        table_hbm.at[pl.ds(vo[j], 1)],
                        rows_o.at[pl.ds(j0 + j, 1)], sem)

            pltpu.make_async_copy(
                table_hbm.at[pl.ds(0, CH2)], rows_e, sem).wait()
            pltpu.make_async_copy(
                table_hbm.at[pl.ds(0, CH2)], rows_o, sem).wait()

            pltpu.sync_copy(rows_e, oe_hbm.at[pl.ds(base, CH2)])
            pltpu.sync_copy(rows_o, oo_hbm.at[pl.ds(base, CH2)])

    return gather_kernel(emb, idx_even, idx_odd)


def _gru_body(xe_hbm, xo_hbm, wih_ref, whh_ref, bih_ref, bhh_ref,
              fcw_ref, fcb_ref, out_ref,
              xe0, xo0, xe1, xo1, h_ref, sem0, sem1):
    pltpu.make_async_copy(xe_hbm.at[0], xe0, sem0).start()
    pltpu.make_async_copy(xo_hbm.at[0], xo0, sem0).start()
    pltpu.make_async_copy(xe_hbm.at[1], xe1, sem1).start()
    pltpu.make_async_copy(xo_hbm.at[1], xo1, sem1).start()
    h_ref[...] = jnp.zeros((B2, PW), jnp.float32)
    wih = wih_ref[...]
    whh = whh_ref[...]
    bih = bih_ref[...]
    bhh = bhh_ref[...]

    def gru_step(xt, h):
        gi = jnp.dot(xt, wih, preferred_element_type=jnp.float32) + bih
        gh = jnp.dot(h, whh, preferred_element_type=jnp.float32) + bhh
        r = jax.nn.sigmoid(gi[:, 0:128] + gh[:, 0:128])
        z = jax.nn.sigmoid(gi[:, 128:256] + gh[:, 128:256])
        n = jnp.tanh(gi[:, 256:384] + r * gh[:, 256:384])
        return (1.0 - z) * n + z * h

    def pair(i, carry):
        t0 = 2 * i
        pltpu.make_async_copy(xe_hbm.at[t0], xe0, sem0).wait()
        pltpu.make_async_copy(xo_hbm.at[t0], xo0, sem0).wait()
        xt = jnp.concatenate([xe0[...], xo0[...]], axis=1)
        h_ref[...] = gru_step(xt, h_ref[...])

        @pl.when(i < (T // 2) - 1)
        def _():
            pltpu.make_async_copy(xe_hbm.at[t0 + 2], xe0, sem0).start()
            pltpu.make_async_copy(xo_hbm.at[t0 + 2], xo0, sem0).start()

        pltpu.make_async_copy(xe_hbm.at[t0 + 1], xe1, sem1).wait()
        pltpu.make_async_copy(xo_hbm.at[t0 + 1], xo1, sem1).wait()
        xt = jnp.concatenate([xe1[...], xo1[...]], axis=1)
        h_ref[...] = gru_step(xt, h_ref[...])

        @pl.when(i < (T // 2) - 1)
        def _():
            pltpu.make_async_copy(xe_hbm.at[t0 + 3], xe1, sem1).start()
            pltpu.make_async_copy(xo_hbm.at[t0 + 3], xo1, sem1).start()

        return carry

    lax.fori_loop(0, T // 2, pair, 0)
    out_ref[...] = (
        jnp.dot(h_ref[...], fcw_ref[...], preferred_element_type=jnp.float32)
        + fcb_ref[...]
    )


def _gru_tc(xs_e, xs_o, wih2, whh2, bih2, bhh2, fcw2, fcb2):
    return pl.pallas_call(
        _gru_body,
        in_specs=[
            pl.BlockSpec(memory_space=pl.ANY),
            pl.BlockSpec(memory_space=pl.ANY),
            pl.BlockSpec(memory_space=pltpu.MemorySpace.VMEM),
            pl.BlockSpec(memory_space=pltpu.MemorySpace.VMEM),
            pl.BlockSpec(memory_space=pltpu.MemorySpace.VMEM),
            pl.BlockSpec(memory_space=pltpu.MemorySpace.VMEM),
            pl.BlockSpec(memory_space=pltpu.MemorySpace.VMEM),
            pl.BlockSpec(memory_space=pltpu.MemorySpace.VMEM),
        ],
        out_specs=pl.BlockSpec(memory_space=pltpu.MemorySpace.VMEM),
        out_shape=jax.ShapeDtypeStruct((B2, PW), jnp.float32),
        scratch_shapes=[
            pltpu.VMEM((B2, EMBED), jnp.float32),
            pltpu.VMEM((B2, EMBED), jnp.float32),
            pltpu.VMEM((B2, EMBED), jnp.float32),
            pltpu.VMEM((B2, EMBED), jnp.float32),
            pltpu.VMEM((B2, PW), jnp.float32),
            pltpu.SemaphoreType.DMA,
            pltpu.SemaphoreType.DMA,
        ],
    )(xs_e, xs_o, wih2, whh2, bih2, bhh2, fcw2, fcb2)


def _pair_weights(wT):
    z = jnp.zeros((EMBED, HIDDEN), wT.dtype)
    blocks = []
    for g in range(3):
        wg = wT[:, g * HIDDEN:(g + 1) * HIDDEN]
        top = jnp.concatenate([wg, z], axis=1)
        bot = jnp.concatenate([z, wg], axis=1)
        blocks.append(jnp.concatenate([top, bot], axis=0))
    return jnp.concatenate(blocks, axis=1)


def _pair_bias(b):
    return jnp.tile(b.reshape(3, 1, HIDDEN), (1, 2, 1)).reshape(1, PG)


def kernel(x, emb, w_ih, w_hh, b_ih, b_hh, fc_w, fc_b):
    xt = x.astype(jnp.int32).T
    idx_even = xt[:, 0::2].reshape(-1)
    idx_odd = xt[:, 1::2].reshape(-1)
    rows_e, rows_o = _gather_sc(emb, idx_even, idx_odd)
    xs_e = rows_e.reshape(T, B2, EMBED)
    xs_o = rows_o.reshape(T, B2, EMBED)

    fcwT = fc_w.T
    z = jnp.zeros((HIDDEN, HIDDEN), fcwT.dtype)
    fcw2 = jnp.concatenate(
        [jnp.concatenate([fcwT, z], axis=1),
         jnp.concatenate([z, fcwT], axis=1)], axis=0)

    out2 = _gru_tc(
        xs_e, xs_o,
        _pair_weights(w_ih.T),
        _pair_weights(w_hh.T),
        _pair_bias(b_ih),
        _pair_bias(b_hh),
        fcw2,
        jnp.tile(fc_b.reshape(1, HIDDEN), (1, 2)),
    )
    return out2.reshape(B, HIDDEN)


# speedup vs baseline: 1.7041x; 1.2346x over previous
import functools

import jax
import jax.numpy as jnp
from jax import lax
from jax.experimental import pallas as pl
from jax.experimental.pallas import tpu as pltpu
from jax.experimental.pallas import tpu_sc as plsc

VOCAB = 1000000
EMBED = 64
HIDDEN = 64
B = 4096
T = 50
G3 = 3 * HIDDEN
PW = 128

NC = 2
NS = 16
NW = NC * NS
CH = 400

RB = 16384


def _format_body(et_ref, out_ref):
    out_ref[:, 0:EMBED] = et_ref[...].T


def _format_table(embT):
    n_steps = (VOCAB + RB - 1) // RB
    return pl.pallas_call(
        _format_body,
        grid=(n_steps,),
        in_specs=[pl.BlockSpec((EMBED, RB), lambda i: (0, i))],
        out_specs=pl.BlockSpec((RB, PW), lambda i: (i, 0)),
        out_shape=jax.ShapeDtypeStruct((VOCAB, PW), jnp.float32),
    )(embT)


def _gather_sc(table, idx_flat):
    TB = idx_flat.shape[0]
    b_per_w = TB // NW
    n_ch = b_per_w // CH
    mesh = plsc.VectorSubcoreMesh(core_axis_name="c", subcore_axis_name="s")

    @functools.partial(
        pl.kernel,
        out_type=jax.ShapeDtypeStruct((TB, PW), jnp.float32),
        mesh=mesh,
        scratch_types=[
            pltpu.VMEM((CH,), jnp.int32),
            pltpu.VMEM((CH, PW), jnp.float32),
            pltpu.SemaphoreType.DMA,
        ],
    )
    def gather_kernel(table_hbm, i_hbm, o_hbm, idx_v, rows_v, sem):
        wid = lax.axis_index("s") * NC + lax.axis_index("c")

        @pl.loop(0, n_ch)
        def _(c):
            base = wid * b_per_w + c * CH
            pltpu.sync_copy(i_hbm.at[pl.ds(base, CH)], idx_v)
            pltpu.async_copy(table_hbm.at[idx_v], rows_v, sem).wait()
            pltpu.sync_copy(rows_v, o_hbm.at[pl.ds(base, CH)])

    return gather_kernel(table, idx_flat)


def _gru_body(xs_hbm, wih_ref, whh_ref, bih_ref, bhh_ref, fcw_ref, fcb_ref,
              out_ref, x0, x1, h_ref, sem0, sem1):
    pltpu.make_async_copy(xs_hbm.at[0], x0, sem0).start()
    pltpu.make_async_copy(xs_hbm.at[1], x1, sem1).start()
    h_ref[...] = jnp.zeros((B, HIDDEN), jnp.float32)
    wih = wih_ref[...]
    whh = whh_ref[...]
    bih = bih_ref[...]
    bhh = bhh_ref[...]

    def gru_step(xt, h):
        gi = jnp.dot(xt, wih, preferred_element_type=jnp.float32) + bih
        gh = jnp.dot(h, whh, preferred_element_type=jnp.float32) + bhh
        r = jax.nn.sigmoid(gi[:, 0:HIDDEN] + gh[:, 0:HIDDEN])
        z = jax.nn.sigmoid(gi[:, HIDDEN:2 * HIDDEN] + gh[:, HIDDEN:2 * HIDDEN])
        n = jnp.tanh(gi[:, 2 * HIDDEN:] + r * gh[:, 2 * HIDDEN:])
        return (1.0 - z) * n + z * h

    def pair(i, carry):
        t0 = 2 * i
        pltpu.make_async_copy(xs_hbm.at[t0], x0, sem0).wait()
        h_ref[...] = gru_step(x0[:, 0:EMBED], h_ref[...])

        @pl.when(i < (T // 2) - 1)
        def _():
            pltpu.make_async_copy(xs_hbm.at[t0 + 2], x0, sem0).start()

        pltpu.make_async_copy(xs_hbm.at[t0 + 1], x1, sem1).wait()
        h_ref[...] = gru_step(x1[:, 0:EMBED], h_ref[...])

        @pl.when(i < (T // 2) - 1)
        def _():
            pltpu.make_async_copy(xs_hbm.at[t0 + 3], x1, sem1).start()

        return carry

    lax.fori_loop(0, T // 2, pair, 0)
    out_ref[...] = (
        jnp.dot(h_ref[...], fcw_ref[...], preferred_element_type=jnp.float32)
        + fcb_ref[...]
    )


def _gru_tc(xs, wihT, whhT, bih, bhh, fcwT, fcb):
    return pl.pallas_call(
        _gru_body,
        in_specs=[
            pl.BlockSpec(memory_space=pl.ANY),
            pl.BlockSpec(memory_space=pltpu.MemorySpace.VMEM),
            pl.BlockSpec(memory_space=pltpu.MemorySpace.VMEM),
            pl.BlockSpec(memory_space=pltpu.MemorySpace.VMEM),
            pl.BlockSpec(memory_space=pltpu.MemorySpace.VMEM),
            pl.BlockSpec(memory_space=pltpu.MemorySpace.VMEM),
            pl.BlockSpec(memory_space=pltpu.MemorySpace.VMEM),
        ],
        out_specs=pl.BlockSpec(memory_space=pltpu.MemorySpace.VMEM),
        out_shape=jax.ShapeDtypeStruct((B, HIDDEN), jnp.float32),
        scratch_shapes=[
            pltpu.VMEM((B, PW), jnp.float32),
            pltpu.VMEM((B, PW), jnp.float32),
            pltpu.VMEM((B, HIDDEN), jnp.float32),
            pltpu.SemaphoreType.DMA,
            pltpu.SemaphoreType.DMA,
        ],
    )(xs, wihT, whhT, bih, bhh, fcwT, fcb)


def kernel(x, emb, w_ih, w_hh, b_ih, b_hh, fc_w, fc_b):
    table = _format_table(emb.T)
    idx = x.astype(jnp.int32).T.reshape(-1)
    rows = _gather_sc(table, idx)
    xs = rows.reshape(T, B, PW)
    return _gru_tc(
        xs,
        w_ih.T,
        w_hh.T,
        b_ih.reshape(1, G3),
        b_hh.reshape(1, G3),
        fc_w.T,
        fc_b.reshape(1, HIDDEN),
    )
